# Initial kernel scaffold; baseline (speedup 1.0000x reference)
import jax
import jax.numpy as jnp
from jax.experimental import pallas as pl


def kernel(x, conv_w, gamma, beta):
    raise NotImplementedError("write your pallas kernel here")


# speedup vs baseline: 4.9680x; 4.9680x over previous
import numpy as np
import jax
import jax.numpy as jnp
from jax.experimental import pallas as pl
from jax.experimental.pallas import tpu as pltpu

_B, _C, _H, _W = 8, 192, 80, 80
_T = _H * _W
_K, _X = 4, 3
_NA = _T // _K * (_K - 1)
_ND = _T // _K
_NBLK = 10
_BN = _NA // _NBLK
_WIN = 128


def _build_static():
    idx = np.arange(_T)
    a_idx = idx[: _NA * (_K - 1)].reshape(-1, _K)[:, : _K - 1].reshape(-1)
    b_idx = idx[_K - 1 :: _K]

    def coords(ix):
        side = int(np.sqrt(ix.shape[0]))
        return np.stack([ix // side, ix % side], -1).astype(np.float32)

    ac, bc = coords(a_idx), coords(b_idx)
    d2 = ((ac[:, None, :] - bc[None, :, :]) ** 2).sum(-1)
    nearest = np.argsort(d2, axis=1, kind="stable")[:, :_X]

    w0 = []
    E = np.zeros((_NBLK, _X, _BN, _WIN), np.float32)
    for j in range(_NBLK):
        blk = nearest[j * _BN : (j + 1) * _BN]
        lo = min((int(blk.min()) // 8) * 8, _ND - _WIN)
        assert int(blk.max()) - lo < _WIN
        w0.append(lo)
        rel = blk - lo
        for xx in range(_X):
            E[j, xx, np.arange(_BN), rel[:, xx]] = 1.0
    return w0, E


_W0, _E = _build_static()
_EJ = jnp.asarray(_E)


def _main_body(x_ref, e_ref, ypre_ref, stats_ref, acc_s, acc_c):
    xb = x_ref[0]
    tr = xb.T
    tokens = tr.reshape(_ND, _K, _C)
    src_all = tokens[:, : _K - 1, :].reshape(_NA, _C)
    dst_all = tokens[:, _K - 1, :]
    dstn = dst_all / jnp.sqrt(jnp.sum(dst_all * dst_all, axis=1, keepdims=True))
    srcn = src_all / jnp.sqrt(jnp.sum(src_all * src_all, axis=1, keepdims=True))
    srcn_bf = srcn.astype(jnp.bfloat16)
    dstn_bf = dstn.astype(jnp.bfloat16)

    acc_s[...] = dst_all
    acc_c[...] = jnp.ones((_ND, 8), jnp.float32)
    ones_col = jnp.ones((_BN, 8), jnp.float32)

    for j in range(_NBLK):
        w0 = _W0[j]
        srcb = src_all[j * _BN : (j + 1) * _BN]
        ab = srcn_bf[j * _BN : (j + 1) * _BN]
        dw = dstn_bf[w0 : w0 + _WIN]
        S = jax.lax.dot_general(
            ab, dw, (((1,), (1,)), ((), ())),
            preferred_element_type=jnp.float32)
        e0 = e_ref[j, 0]
        e1 = e_ref[j, 1]
        e2 = e_ref[j, 2]
        s0 = jnp.sum(S * e0, axis=1, keepdims=True)
        s1 = jnp.sum(S * e1, axis=1, keepdims=True)
        s2 = jnp.sum(S * e2, axis=1, keepdims=True)
        gt10 = s1 > s0
        is2 = s2 > jnp.maximum(s0, s1)
        not2 = jnp.logical_not(is2)
        f0 = jnp.where(jnp.logical_and(not2, jnp.logical_not(gt10)), 1.0, 0.0)
        f1 = jnp.where(jnp.logical_and(not2, gt10), 1.0, 0.0)
        f2 = jnp.where(is2, 1.0, 0.0)
        onehot = f0 * e0 + f1 * e1 + f2 * e2
        sums = jax.lax.dot_general(
            onehot, srcb, (((0,), (0,)), ((), ())),
            preferred_element_type=jnp.float32)
        cnts = jax.lax.dot_general(
            onehot, ones_col, (((0,), (0,)), ((), ())),
            preferred_element_type=jnp.float32)
        acc_s[w0 : w0 + _WIN, :] += sums
        acc_c[w0 : w0 + _WIN, :] += cnts

    merged = acc_s[...] / acc_c[:, 0:1]
    return merged


def _fused_main(x_ref, e_ref, w_ref, ypre_ref, stats_ref, acc_s, acc_c):
    merged = _main_body(x_ref, e_ref, ypre_ref, stats_ref, acc_s, acc_c)
    y = jax.lax.dot_general(
        merged, w_ref[...], (((1,), (1,)), ((), ())),
        preferred_element_type=jnp.float32)
    yt = y.T
    ypre_ref[0] = yt
    ssum = jnp.sum(yt, axis=1, keepdims=True)
    ssq = jnp.sum(yt * yt, axis=1, keepdims=True)
    stats_ref[0] = jnp.concatenate(
        [ssum, ssq, jnp.zeros((_C, 6), jnp.float32)], axis=1)


def _bn_silu(ypre_ref, stats_ref, g_ref, b_ref, out_ref):
    tot = jnp.sum(stats_ref[...], axis=0)
    n = float(_B * _ND)
    mean = tot[:, 0:1] / n
    var = tot[:, 1:2] / n - mean * mean
    invs = 1.0 / jnp.sqrt(var + 1e-3)
    y = ypre_ref[0]
    yn = (y - mean) * invs * g_ref[...] + b_ref[...]
    out_ref[0] = yn * jax.nn.sigmoid(yn)


def kernel(x, conv_w, gamma, beta):
    x3 = x.reshape(_B, _C, _T)
    ypre, stats = pl.pallas_call(
        _fused_main,
        grid=(_B,),
        in_specs=[
            pl.BlockSpec((1, _C, _T), lambda b: (b, 0, 0)),
            pl.BlockSpec((_NBLK, _X, _BN, _WIN), lambda b: (0, 0, 0, 0)),
            pl.BlockSpec((_C, _C), lambda b: (0, 0)),
        ],
        out_specs=[
            pl.BlockSpec((1, _C, _ND), lambda b: (b, 0, 0)),
            pl.BlockSpec((1, _C, 8), lambda b: (b, 0, 0)),
        ],
        out_shape=[
            jax.ShapeDtypeStruct((_B, _C, _ND), jnp.float32),
            jax.ShapeDtypeStruct((_B, _C, 8), jnp.float32),
        ],
        scratch_shapes=[
            pltpu.VMEM((_ND, _C), jnp.float32),
            pltpu.VMEM((_ND, 8), jnp.float32),
        ],
    )(x3, _EJ, conv_w)

    out = pl.pallas_call(
        _bn_silu,
        grid=(_B,),
        in_specs=[
            pl.BlockSpec((1, _C, _ND), lambda b: (b, 0, 0)),
            pl.BlockSpec((_B, _C, 8), lambda b: (0, 0, 0)),
            pl.BlockSpec((_C, 1), lambda b: (0, 0)),
            pl.BlockSpec((_C, 1), lambda b: (0, 0)),
        ],
        out_specs=pl.BlockSpec((1, _C, _ND), lambda b: (b, 0, 0)),
        out_shape=jax.ShapeDtypeStruct((_B, _C, _ND), jnp.float32),
    )(ypre, stats, gamma.reshape(_C, 1), beta.reshape(_C, 1))

    return out.reshape(_B, _C, _H // 2, _W // 2)
